# baseline (speedup 1.0000x reference)
import functools

import jax
import jax.numpy as jnp
from jax import lax
from jax.experimental import pallas as pl
from jax.experimental.pallas import tpu as pltpu
from jax.experimental.pallas import tpu_sc as plsc

DICT_LEN = 1000000
QUESTION_DIM = 32
BATCH = 16384
HIST = 50
TOTAL = BATCH * HIST

NUM_CORES = 2
NUM_SUBCORES = 16
NW = NUM_CORES * NUM_SUBCORES
PER_W = TOTAL // NW
CHUNK = 128
NCHUNK = PER_W // CHUNK
NBUF = 8
PREFETCH = 4
NGRP = NCHUNK // NBUF

_MESH = plsc.VectorSubcoreMesh(core_axis_name="c", subcore_axis_name="s")


@functools.partial(
    pl.kernel,
    mesh=_MESH,
    compiler_params=pltpu.CompilerParams(use_tc_tiling_on_sc=False),
    out_type=jax.ShapeDtypeStruct((TOTAL, QUESTION_DIM), jnp.float32),
    scratch_types=[
        pltpu.VMEM((NCHUNK, CHUNK), jnp.int32),
        pltpu.VMEM((NBUF, CHUNK, QUESTION_DIM), jnp.float32),
        [pltpu.SemaphoreType.DMA] * NBUF,
        [pltpu.SemaphoreType.DMA] * NBUF,
    ],
)
def _gather_kernel(table_hbm, idx_hbm, out_hbm, idx_v, rows_v, gsems, osems):
    wid = lax.axis_index("s") * NUM_CORES + lax.axis_index("c")
    base = wid * PER_W
    pltpu.sync_copy(idx_hbm.at[wid], idx_v)

    def fire_gather(chunk, buf):
        pltpu.async_copy(
            table_hbm.at[idx_v.at[chunk]], rows_v.at[buf], gsems[buf]
        )

    def wait_gather(buf):
        pltpu.make_async_copy(
            table_hbm.at[pl.ds(0, CHUNK)], rows_v.at[buf], gsems[buf]
        ).wait()

    def fire_write(chunk, buf):
        pltpu.async_copy(
            rows_v.at[buf], out_hbm.at[pl.ds(base + chunk * CHUNK, CHUNK)],
            osems[buf],
        )

    def wait_write(buf):
        pltpu.make_async_copy(
            rows_v.at[buf], out_hbm.at[pl.ds(base, CHUNK)], osems[buf]
        ).wait()

    for b in range(PREFETCH):
        fire_gather(b, b)

    def group_body(g, carry):
        for b in range(NBUF):
            j = g * NBUF + b
            f = (b + PREFETCH) % NBUF
            jf = j + PREFETCH

            @pl.when(jf >= NBUF)
            def _():
                wait_write(f)

            @pl.when(jf < NCHUNK)
            def _():
                fire_gather(jf, f)

            wait_gather(b)
            fire_write(j, b)
        return carry

    lax.fori_loop(0, NGRP, group_body, 0)

    for b in range(NBUF - PREFETCH):
        buf = (b + PREFETCH) % NBUF
        wait_write(buf)


def _eye32():
    r = lax.broadcasted_iota(jnp.int32, (QUESTION_DIM, QUESTION_DIM), 0)
    c = lax.broadcasted_iota(jnp.int32, (QUESTION_DIM, QUESTION_DIM), 1)
    return (r == c).astype(jnp.float32)


_TBL_COLS = 8192


def _tbl_xform_body(x_ref, o_ref):
    y = lax.dot_general(
        x_ref[...], _eye32(), (((0,), (0,)), ((), ())),
        preferred_element_type=jnp.float32,
        precision=lax.Precision.HIGHEST,
    )
    y3 = y.reshape(_TBL_COLS // 4, 4, QUESTION_DIM)
    for q in range(4):
        o_ref[:, q * QUESTION_DIM:(q + 1) * QUESTION_DIM] = y3[:, q, :]


_tbl_xform = pl.pallas_call(
    _tbl_xform_body,
    grid=(pl.cdiv(DICT_LEN, _TBL_COLS),),
    in_specs=[pl.BlockSpec((QUESTION_DIM, _TBL_COLS), lambda i: (0, i))],
    out_specs=pl.BlockSpec((_TBL_COLS // 4, 128), lambda i: (i, 0)),
    out_shape=jax.ShapeDtypeStruct((DICT_LEN // 4, 128), jnp.float32),
)


_BC = 4096
_NBB = BATCH // _BC


def _out_xform_body(x_ref, o_ref):
    x128 = x_ref[...]
    parts = [x128[:, q * QUESTION_DIM:(q + 1) * QUESTION_DIM] for q in range(4)]
    x = jnp.stack(parts, axis=1).reshape(_BC, QUESTION_DIM)
    o_ref[0] = lax.dot_general(
        _eye32(), x, (((1,), (1,)), ((), ())),
        preferred_element_type=jnp.float32,
        precision=lax.Precision.HIGHEST,
    )


_out_xform = pl.pallas_call(
    _out_xform_body,
    grid=(HIST, _NBB),
    in_specs=[pl.BlockSpec(
        (_BC // 4, 128), lambda h, j: (h * _NBB + j, 0))],
    out_specs=pl.BlockSpec(
        (1, QUESTION_DIM, _BC), lambda h, j: (h, 0, j)),
    out_shape=jax.ShapeDtypeStruct((HIST, QUESTION_DIM, BATCH), jnp.float32),
)


def kernel(question, weight):
    idx = question.T.reshape(NW, NCHUNK, CHUNK).astype(jnp.int32)
    table = _tbl_xform(weight.T).reshape(DICT_LEN, QUESTION_DIM)
    flat = _gather_kernel(table, idx)
    out = _out_xform(flat.reshape(TOTAL // 4, 128))
    return out.transpose(2, 0, 1)


# speedup vs baseline: 1.2945x; 1.2945x over previous
import functools

import jax
import jax.numpy as jnp
from jax import lax
from jax.experimental import pallas as pl
from jax.experimental.pallas import tpu as pltpu
from jax.experimental.pallas import tpu_sc as plsc

DICT_LEN = 1000000
QUESTION_DIM = 32
BATCH = 16384
HIST = 50
TOTAL = BATCH * HIST

NUM_CORES = 2
NUM_SUBCORES = 16
NW = NUM_CORES * NUM_SUBCORES
PER_W = TOTAL // NW
CHUNK = 128
NCHUNK = PER_W // CHUNK
NBUF = 8
PREFETCH = 4
NGRP = NCHUNK // NBUF

_MESH = plsc.VectorSubcoreMesh(core_axis_name="c", subcore_axis_name="s")


@functools.partial(
    pl.kernel,
    mesh=_MESH,
    compiler_params=pltpu.CompilerParams(use_tc_tiling_on_sc=False),
    out_type=jax.ShapeDtypeStruct((TOTAL, QUESTION_DIM), jnp.float32),
    scratch_types=[
        pltpu.VMEM((NCHUNK, CHUNK), jnp.int32),
        pltpu.VMEM((NBUF, CHUNK, QUESTION_DIM), jnp.float32),
        [pltpu.SemaphoreType.DMA] * NBUF,
        [pltpu.SemaphoreType.DMA] * NBUF,
    ],
)
def _gather_kernel(table_hbm, idx_hbm, out_hbm, idx_v, rows_v, gsems, osems):
    wid = lax.axis_index("s") * NUM_CORES + lax.axis_index("c")
    base = wid * PER_W
    pltpu.sync_copy(idx_hbm.at[wid], idx_v)

    def fire_gather(chunk, buf):
        pltpu.async_copy(
            table_hbm.at[idx_v.at[chunk]], rows_v.at[buf], gsems[buf]
        )

    def wait_gather(buf):
        pltpu.make_async_copy(
            table_hbm.at[pl.ds(0, CHUNK)], rows_v.at[buf], gsems[buf]
        ).wait()

    def fire_write(chunk, buf):
        pltpu.async_copy(
            rows_v.at[buf], out_hbm.at[pl.ds(base + chunk * CHUNK, CHUNK)],
            osems[buf],
        )

    def wait_write(buf):
        pltpu.make_async_copy(
            rows_v.at[buf], out_hbm.at[pl.ds(base, CHUNK)], osems[buf]
        ).wait()

    for b in range(PREFETCH):
        fire_gather(b, b)

    def group_body(g, carry):
        for b in range(NBUF):
            j = g * NBUF + b
            f = (b + PREFETCH) % NBUF
            jf = j + PREFETCH

            @pl.when(jf >= NBUF)
            def _():
                wait_write(f)

            @pl.when(jf < NCHUNK)
            def _():
                fire_gather(jf, f)

            wait_gather(b)
            fire_write(j, b)
        return carry

    lax.fori_loop(0, NGRP, group_body, 0)

    for b in range(NBUF - PREFETCH):
        buf = (b + PREFETCH) % NBUF
        wait_write(buf)


def _eye32():
    r = lax.broadcasted_iota(jnp.int32, (QUESTION_DIM, QUESTION_DIM), 0)
    c = lax.broadcasted_iota(jnp.int32, (QUESTION_DIM, QUESTION_DIM), 1)
    return (r == c).astype(jnp.float32)


_TBL_COLS = 8192
_TBL_M = _TBL_COLS // 4
_TBL_BLOCKS = (DICT_LEN + _TBL_COLS - 1) // _TBL_COLS
_TBL_ROWS = _TBL_BLOCKS * _TBL_M


def _tbl_xform_body(x_ref, o_ref):
    x = x_ref[...]
    for q in range(4):
        xq = x[:, q * _TBL_M:(q + 1) * _TBL_M]
        yq = lax.dot_general(
            xq, _eye32(), (((0,), (0,)), ((), ())),
            preferred_element_type=jnp.float32,
            precision=lax.Precision.HIGHEST,
        )
        o_ref[:, q * QUESTION_DIM:(q + 1) * QUESTION_DIM] = yq


_tbl_xform = pl.pallas_call(
    _tbl_xform_body,
    grid=(_TBL_BLOCKS,),
    in_specs=[pl.BlockSpec((QUESTION_DIM, _TBL_COLS), lambda i: (0, i))],
    out_specs=pl.BlockSpec((_TBL_M, 128), lambda i: (i, 0)),
    out_shape=jax.ShapeDtypeStruct((_TBL_ROWS, 128), jnp.float32),
)


_BC = 4096
_BM = _BC // 4
_NBB = BATCH // _BC


def _out_xform_body(x_ref, o_ref):
    x = x_ref[...]
    for q in range(4):
        xq = x[:, q * QUESTION_DIM:(q + 1) * QUESTION_DIM]
        yq = lax.dot_general(
            _eye32(), xq, (((1,), (1,)), ((), ())),
            preferred_element_type=jnp.float32,
            precision=lax.Precision.HIGHEST,
        )
        o_ref[0, :, q * _BM:(q + 1) * _BM] = yq


_out_xform = pl.pallas_call(
    _out_xform_body,
    grid=(HIST, _NBB),
    in_specs=[pl.BlockSpec(
        (_BM, 128), lambda h, j: (h * _NBB + j, 0))],
    out_specs=pl.BlockSpec(
        (1, QUESTION_DIM, _BC), lambda h, j: (h, 0, j)),
    out_shape=jax.ShapeDtypeStruct((HIST, QUESTION_DIM, BATCH), jnp.float32),
)


def kernel(question, weight):
    qt = question.T.reshape(HIST, _NBB, 4, _BM)
    idx = qt.transpose(0, 1, 3, 2).reshape(NW, NCHUNK, CHUNK).astype(jnp.int32)
    blk = idx // _TBL_COLS
    off = idx % _TBL_COLS
    idx = blk * _TBL_COLS + 4 * (off % _TBL_M) + off // _TBL_M
    table = _tbl_xform(weight.T).reshape(_TBL_ROWS * 4, QUESTION_DIM)
    flat = _gather_kernel(table, idx)
    out = _out_xform(flat.reshape(TOTAL // 4, 128))
    return out.transpose(2, 0, 1)
